# baseline (speedup 1.0000x reference)
import jax
import jax.numpy as jnp
from jax.experimental import pallas as pl
from jax.experimental.pallas import tpu as pltpu

Q = 1024
D = 64
KDB = 100000
KPAD = 100352
CHUNK = 2048
QB = 256
TOPK = 50


def _ksq_body(k_ref, out_ref):
    kblk = k_ref[...]
    out_ref[...] = jnp.sum(kblk * kblk, axis=1, keepdims=True)


def _ksq(kpad):
    return pl.pallas_call(
        _ksq_body,
        grid=(KPAD // CHUNK,),
        in_specs=[pl.BlockSpec((CHUNK, D), lambda kc: (kc, 0))],
        out_specs=pl.BlockSpec((CHUNK, 1), lambda kc: (kc, 0)),
        out_shape=jax.ShapeDtypeStruct((KPAD, 1), jnp.float32),
    )(kpad)


def _dist_body(q_ref, k_ref, qsq_ref, ksq_ref, out_ref):
    qblk = q_ref[...]
    kblk = k_ref[...]
    dots = jax.lax.dot_general(
        qblk.astype(jnp.bfloat16), kblk.astype(jnp.bfloat16),
        (((1,), (1,)), ((), ())),
        preferred_element_type=jnp.float32)
    out_ref[...] = (qsq_ref[...] + ksq_ref[...]) - 2.0 * dots


def _dists(q, kpad, qsq, ksq_row):
    return pl.pallas_call(
        _dist_body,
        grid=(KPAD // CHUNK, Q // QB),
        in_specs=[
            pl.BlockSpec((QB, D), lambda kc, qb: (qb, 0)),
            pl.BlockSpec((CHUNK, D), lambda kc, qb: (kc, 0)),
            pl.BlockSpec((QB, 1), lambda kc, qb: (qb, 0)),
            pl.BlockSpec((1, CHUNK), lambda kc, qb: (0, kc)),
        ],
        out_specs=pl.BlockSpec((QB, CHUNK), lambda kc, qb: (qb, kc)),
        out_shape=jax.ShapeDtypeStruct((Q, KPAD), jnp.float32),
    )(q, kpad, qsq, ksq_row)


def kernel(input_embeddings, index_embeddings, top_k):
    kpad = jnp.concatenate(
        [index_embeddings,
         jnp.full((KPAD - KDB, D), 1e4, jnp.float32)], axis=0)
    qsq = jnp.sum(input_embeddings * input_embeddings, axis=1, keepdims=True)
    ksq_row = _ksq(kpad).reshape(1, KPAD)
    dists = _dists(input_embeddings, kpad, qsq, ksq_row)
    _, idx = jax.lax.top_k(-dists[:, :KDB], TOPK)
    return idx


# speedup vs baseline: 3.3191x; 3.3191x over previous
import functools

import jax
import jax.numpy as jnp
from jax import lax
from jax.experimental import pallas as pl
from jax.experimental.pallas import tpu as pltpu
from jax.experimental.pallas import tpu_sc as plsc

Q = 1024
D = 64
KDB = 100000
KPAD = 100352
CHUNK = 2048
SUB = 512
NSUB = KPAD // SUB
NSUBPAD = 256
QB = 256
TOPK = 50
B = 256
BPAD = B + 16
W = 12544
NW = KPAD // W
QPW = 32
INF = float('inf')


def _ksq_body(k_ref, out_ref):
    kblk = k_ref[...]
    out_ref[...] = jnp.sum(kblk * kblk, axis=1, keepdims=True)


def _ksq(kpad):
    return pl.pallas_call(
        _ksq_body,
        grid=(KPAD // CHUNK,),
        in_specs=[pl.BlockSpec((CHUNK, D), lambda kc: (kc, 0))],
        out_specs=pl.BlockSpec((CHUNK, 1), lambda kc: (kc, 0)),
        out_shape=jax.ShapeDtypeStruct((KPAD, 1), jnp.float32),
    )(kpad)


def _dist_body(q_ref, k_ref, qsq_ref, ksq_ref, out_ref, cm_ref):
    qblk = q_ref[...]
    kblk = k_ref[...]
    dots = jax.lax.dot_general(
        qblk.astype(jnp.bfloat16), kblk.astype(jnp.bfloat16),
        (((1,), (1,)), ((), ())),
        preferred_element_type=jnp.float32)
    dists = (qsq_ref[...] + ksq_ref[...]) - 2.0 * dots
    out_ref[...] = dists
    mins = [jnp.min(dists[:, s * SUB:(s + 1) * SUB], axis=1, keepdims=True)
            for s in range(CHUNK // SUB)]
    cm_ref[0] = jnp.concatenate(mins, axis=1)


def _dists(q, kpad, qsq, ksq_row):
    return pl.pallas_call(
        _dist_body,
        grid=(KPAD // CHUNK, Q // QB),
        in_specs=[
            pl.BlockSpec((QB, D), lambda kc, qb: (qb, 0)),
            pl.BlockSpec((CHUNK, D), lambda kc, qb: (kc, 0)),
            pl.BlockSpec((QB, 1), lambda kc, qb: (qb, 0)),
            pl.BlockSpec((1, CHUNK), lambda kc, qb: (0, kc)),
        ],
        out_specs=[
            pl.BlockSpec((QB, CHUNK), lambda kc, qb: (qb, kc)),
            pl.BlockSpec((1, QB, CHUNK // SUB), lambda kc, qb: (kc, qb, 0)),
        ],
        out_shape=[
            jax.ShapeDtypeStruct((Q, KPAD), jnp.float32),
            jax.ShapeDtypeStruct((KPAD // CHUNK, Q, CHUNK // SUB),
                                 jnp.float32),
        ],
    )(q, kpad, qsq, ksq_row)


def _lo_body(cm_ref, out_ref):
    a = cm_ref[...]
    m = jnp.min(a, axis=1, keepdims=True)
    for _ in range(TOPK - 1):
        a = jnp.where(a == m, INF, a)
        m = jnp.min(a, axis=1, keepdims=True)
    out_ref[...] = m


def _lo(cm):
    return pl.pallas_call(
        _lo_body,
        grid=(Q // QB,),
        in_specs=[pl.BlockSpec((QB, NSUBPAD), lambda qb: (qb, 0))],
        out_specs=pl.BlockSpec((QB, 1), lambda qb: (qb, 0)),
        out_shape=jax.ShapeDtypeStruct((Q, 1), jnp.float32),
    )(cm)


def _sc_compact(dists, upb):
    info = plsc.get_sparse_core_info()
    mesh = plsc.VectorSubcoreMesh(core_axis_name="c", subcore_axis_name="s")

    @functools.partial(
        pl.kernel, mesh=mesh,
        compiler_params=pltpu.CompilerParams(needs_layout_passes=False),
        out_type=[
            jax.ShapeDtypeStruct((Q, B), jnp.float32),
            jax.ShapeDtypeStruct((Q, B), jnp.int32),
        ],
        scratch_types=[
            pltpu.VMEM((W,), jnp.float32),
            pltpu.VMEM((BPAD,), jnp.float32),
            pltpu.VMEM((BPAD,), jnp.int32),
            pltpu.VMEM((16,), jnp.float32),
            pltpu.SemaphoreType.DMA,
        ],
    )
    def body(s_hbm, up_hbm, cv_hbm, ci_hbm, win_v, cv_v, ci_v, up_v, sem):
        wid = lax.axis_index("s") * info.num_cores + lax.axis_index("c")
        q0 = wid * QPW

        def per_query(t, _):
            qi = q0 + t
            pltpu.sync_copy(up_hbm.at[qi], up_v)
            upb = up_v[...]
            neg = jnp.full((16,), INF, jnp.float32)
            zer = jnp.zeros((16,), jnp.int32)

            def init(b16, _):
                cv_v[pl.ds(b16 * 16, 16)] = neg
                ci_v[pl.ds(b16 * 16, 16)] = zer
                return 0

            lax.fori_loop(0, BPAD // 16, init, 0)

            def per_window(win, off):
                pltpu.sync_copy(s_hbm.at[qi, pl.ds(win * W, W)], win_v)

                def scan(i, off):
                    v = win_v[pl.ds(i * 16, 16)]
                    m = v <= upb
                    idxv = lax.iota(jnp.int32, 16) + (win * W + i * 16)
                    c = plsc.cumsum(jnp.where(m, 1, 0).astype(jnp.int32))
                    pos = off + c - 1
                    plsc.store_scatter(cv_v, [pos], v, mask=m)
                    plsc.store_scatter(ci_v, [pos], idxv, mask=m)
                    return jnp.minimum(off + jnp.max(c), B)

                return lax.fori_loop(0, W // 16, scan, off)

            lax.fori_loop(0, NW, per_window, jnp.int32(0))
            pltpu.sync_copy(cv_v.at[pl.ds(0, B)], cv_hbm.at[qi])
            pltpu.sync_copy(ci_v.at[pl.ds(0, B)], ci_hbm.at[qi])
            return 0

        lax.fori_loop(0, QPW, per_query, 0)

    return body(dists, upb)


def _final_body(cv_ref, ci_ref, out_ref):
    a = cv_ref[...]
    idx = ci_ref[...]
    big = 2147483647
    outs = []
    for _ in range(TOPK):
        m = jnp.min(a, axis=1, keepdims=True)
        sel = a == m
        cand = jnp.min(jnp.where(sel, idx, big), axis=1, keepdims=True)
        outs.append(cand)
        a = jnp.where(sel & (idx == cand), INF, a)
    out_ref[...] = jnp.concatenate(outs, axis=1)


def _final(cv, ci):
    return pl.pallas_call(
        _final_body,
        grid=(Q // QB,),
        in_specs=[
            pl.BlockSpec((QB, B), lambda qb: (qb, 0)),
            pl.BlockSpec((QB, B), lambda qb: (qb, 0)),
        ],
        out_specs=pl.BlockSpec((QB, TOPK), lambda qb: (qb, 0)),
        out_shape=jax.ShapeDtypeStruct((Q, TOPK), jnp.int32),
    )(cv, ci)


def kernel(input_embeddings, index_embeddings, top_k):
    kpad = jnp.concatenate(
        [index_embeddings,
         jnp.full((KPAD - KDB, D), 1e4, jnp.float32)], axis=0)
    qsq = jnp.sum(input_embeddings * input_embeddings, axis=1, keepdims=True)
    ksq_row = _ksq(kpad).reshape(1, KPAD)
    dists, cm3 = _dists(input_embeddings, kpad, qsq, ksq_row)
    cm = jnp.transpose(cm3, (1, 0, 2)).reshape(Q, NSUB)
    cm = jnp.concatenate(
        [cm, jnp.full((Q, NSUBPAD - NSUB), jnp.inf, jnp.float32)], axis=1)
    up = _lo(cm)
    upb = jnp.broadcast_to(up, (Q, 16))
    cv, ci = _sc_compact(dists, upb)
    return _final(cv, ci)


# speedup vs baseline: 8.5988x; 2.5907x over previous
import functools

import jax
import jax.numpy as jnp
from jax import lax
from jax.experimental import pallas as pl
from jax.experimental.pallas import tpu as pltpu
from jax.experimental.pallas import tpu_sc as plsc

Q = 1024
D = 64
KDB = 100000
KPAD = 100352
CHUNK = 2048
SUB = 512
NSUB = KPAD // SUB
NSUBPAD = 256
QB = 256
TOPK = 50
B = 256
BPAD = B + 16
W2 = 50176
QPW = 32
INF = float('inf')


def _ksq_body(k_ref, out_ref):
    kblk = k_ref[...]
    out_ref[...] = jnp.sum(kblk * kblk, axis=1, keepdims=True)


def _ksq(kpad):
    return pl.pallas_call(
        _ksq_body,
        grid=(KPAD // CHUNK,),
        in_specs=[pl.BlockSpec((CHUNK, D), lambda kc: (kc, 0))],
        out_specs=pl.BlockSpec((CHUNK, 1), lambda kc: (kc, 0)),
        out_shape=jax.ShapeDtypeStruct((KPAD, 1), jnp.float32),
    )(kpad)


def _dist_body(q_ref, k_ref, qsq_ref, ksq_ref, out_ref, cm_ref):
    qblk = q_ref[...]
    kblk = k_ref[...]
    dots = jax.lax.dot_general(
        qblk.astype(jnp.bfloat16), kblk.astype(jnp.bfloat16),
        (((1,), (1,)), ((), ())),
        preferred_element_type=jnp.float32)
    dists = (qsq_ref[...] + ksq_ref[...]) - 2.0 * dots
    out_ref[...] = dists
    mins = [jnp.min(dists[:, s * SUB:(s + 1) * SUB], axis=1, keepdims=True)
            for s in range(CHUNK // SUB)]
    cm_ref[0] = jnp.concatenate(mins, axis=1)


def _dists(q, kpad, qsq, ksq_row):
    return pl.pallas_call(
        _dist_body,
        grid=(KPAD // CHUNK, Q // QB),
        in_specs=[
            pl.BlockSpec((QB, D), lambda kc, qb: (qb, 0)),
            pl.BlockSpec((CHUNK, D), lambda kc, qb: (kc, 0)),
            pl.BlockSpec((QB, 1), lambda kc, qb: (qb, 0)),
            pl.BlockSpec((1, CHUNK), lambda kc, qb: (0, kc)),
        ],
        out_specs=[
            pl.BlockSpec((QB, CHUNK), lambda kc, qb: (qb, kc)),
            pl.BlockSpec((1, QB, CHUNK // SUB), lambda kc, qb: (kc, qb, 0)),
        ],
        out_shape=[
            jax.ShapeDtypeStruct((Q, KPAD), jnp.float32),
            jax.ShapeDtypeStruct((KPAD // CHUNK, Q, CHUNK // SUB),
                                 jnp.float32),
        ],
    )(q, kpad, qsq, ksq_row)


def _lo_body(cm_ref, out_ref):
    a = cm_ref[...]
    m = jnp.min(a, axis=1, keepdims=True)
    for _ in range(TOPK - 1):
        a = jnp.where(a == m, INF, a)
        m = jnp.min(a, axis=1, keepdims=True)
    out_ref[...] = m


def _lo(cm):
    return pl.pallas_call(
        _lo_body,
        grid=(Q // QB,),
        in_specs=[pl.BlockSpec((QB, NSUBPAD), lambda qb: (qb, 0))],
        out_specs=pl.BlockSpec((QB, 1), lambda qb: (qb, 0)),
        out_shape=jax.ShapeDtypeStruct((Q, 1), jnp.float32),
    )(cm)


def _sc_compact(dists, upb):
    info = plsc.get_sparse_core_info()
    mesh = plsc.VectorSubcoreMesh(core_axis_name="c", subcore_axis_name="s")

    @functools.partial(
        pl.kernel, mesh=mesh,
        compiler_params=pltpu.CompilerParams(needs_layout_passes=False),
        out_type=[
            jax.ShapeDtypeStruct((Q, B), jnp.float32),
            jax.ShapeDtypeStruct((Q, B), jnp.int32),
        ],
        scratch_types=[
            pltpu.VMEM((2, W2), jnp.float32),
            pltpu.VMEM((BPAD,), jnp.float32),
            pltpu.VMEM((BPAD,), jnp.int32),
            pltpu.VMEM((16,), jnp.float32),
            pltpu.SemaphoreType.DMA,
            pltpu.SemaphoreType.DMA,
        ],
    )
    def body(s_hbm, up_hbm, cv_hbm, ci_hbm, win_v, cv_v, ci_v, up_v,
             sem0, sem1):
        wid = lax.axis_index("s") * info.num_cores + lax.axis_index("c")
        q0 = wid * QPW
        sems = (sem0, sem1)

        def issue(qi, win, buf):
            pltpu.async_copy(
                s_hbm.at[qi, pl.ds(win * W2, W2)], win_v.at[buf], sems[buf])

        def wait(buf):
            pltpu.make_async_copy(
                s_hbm.at[q0, pl.ds(0, W2)], win_v.at[buf], sems[buf]).wait()

        def scan_win(buf, win, off, upb):
            def group(g, off):
                base = g * 128
                mn = win_v[buf, pl.ds(base, 16)]
                for u in range(1, 8):
                    mn = jnp.minimum(mn, win_v[buf, pl.ds(base + u * 16, 16)])
                hit = plsc.all_reduce_population_count(mn <= upb)[0] > 0

                def slow(off):
                    for u in range(8):
                        v = win_v[buf, pl.ds(base + u * 16, 16)]
                        m = v <= upb
                        idxv = lax.iota(jnp.int32, 16) + (
                            win * W2 + base + u * 16)
                        c = plsc.cumsum(jnp.where(m, 1, 0).astype(jnp.int32))
                        pos = off + c - 1
                        plsc.store_scatter(cv_v, [pos], v, mask=m)
                        plsc.store_scatter(ci_v, [pos], idxv, mask=m)
                        off = jnp.minimum(off + jnp.max(c), B)
                    return off

                return lax.cond(hit, slow, lambda off: off, off)

            return lax.fori_loop(0, W2 // 128, group, off)

        issue(q0, 0, 0)

        def per_query(t, _):
            qi = q0 + t
            pltpu.sync_copy(up_hbm.at[qi], up_v)
            upb = up_v[...]
            neg = jnp.full((16,), INF, jnp.float32)
            zer = jnp.zeros((16,), jnp.int32)

            def init(b16, _):
                cv_v[pl.ds(b16 * 16, 16)] = neg
                ci_v[pl.ds(b16 * 16, 16)] = zer
                return 0

            lax.fori_loop(0, BPAD // 16, init, 0)

            issue(qi, 1, 1)
            wait(0)
            off = scan_win(0, 0, jnp.int32(0), upb)

            @pl.when(t < QPW - 1)
            def _():
                issue(qi + 1, 0, 0)

            wait(1)
            off = scan_win(1, 1, off, upb)
            pltpu.sync_copy(cv_v.at[pl.ds(0, B)], cv_hbm.at[qi])
            pltpu.sync_copy(ci_v.at[pl.ds(0, B)], ci_hbm.at[qi])
            return 0

        lax.fori_loop(0, QPW, per_query, 0)

    return body(dists, upb)


def _final_body(cv_ref, ci_ref, out_ref):
    a = cv_ref[...]
    idx = ci_ref[...]
    big = 2147483647
    outs = []
    for _ in range(TOPK):
        m = jnp.min(a, axis=1, keepdims=True)
        sel = a == m
        cand = jnp.min(jnp.where(sel, idx, big), axis=1, keepdims=True)
        outs.append(cand)
        a = jnp.where(sel & (idx == cand), INF, a)
    out_ref[...] = jnp.concatenate(outs, axis=1)


def _final(cv, ci):
    return pl.pallas_call(
        _final_body,
        grid=(Q // QB,),
        in_specs=[
            pl.BlockSpec((QB, B), lambda qb: (qb, 0)),
            pl.BlockSpec((QB, B), lambda qb: (qb, 0)),
        ],
        out_specs=pl.BlockSpec((QB, TOPK), lambda qb: (qb, 0)),
        out_shape=jax.ShapeDtypeStruct((Q, TOPK), jnp.int32),
    )(cv, ci)


def kernel(input_embeddings, index_embeddings, top_k):
    kpad = jnp.concatenate(
        [index_embeddings,
         jnp.full((KPAD - KDB, D), 1e4, jnp.float32)], axis=0)
    qsq = jnp.sum(input_embeddings * input_embeddings, axis=1, keepdims=True)
    ksq_row = _ksq(kpad).reshape(1, KPAD)
    dists, cm3 = _dists(input_embeddings, kpad, qsq, ksq_row)
    cm = jnp.transpose(cm3, (1, 0, 2)).reshape(Q, NSUB)
    cm = jnp.concatenate(
        [cm, jnp.full((Q, NSUBPAD - NSUB), jnp.inf, jnp.float32)], axis=1)
    up = _lo(cm)
    upb = jnp.broadcast_to(up, (Q, 16))
    cv, ci = _sc_compact(dists, upb)
    return _final(cv, ci)


# speedup vs baseline: 11.4461x; 1.3311x over previous
import functools

import jax
import jax.numpy as jnp
from jax import lax
from jax.experimental import pallas as pl
from jax.experimental.pallas import tpu as pltpu
from jax.experimental.pallas import tpu_sc as plsc

Q = 1024
D = 64
KDB = 100000
KPAD = 100352
CHUNK = 2048
SUB = 512
NSUB = KPAD // SUB
NSUBPAD = 256
QB = 256
TOPK = 50
B = 256
BPAD = B + 16
NG16 = KPAD // 16
QPW = 32
INF = float('inf')


def _ksq_body(k_ref, out_ref):
    kblk = k_ref[...]
    out_ref[...] = jnp.sum(kblk * kblk, axis=1, keepdims=True)


def _ksq(kpad):
    return pl.pallas_call(
        _ksq_body,
        grid=(KPAD // CHUNK,),
        in_specs=[pl.BlockSpec((CHUNK, D), lambda kc: (kc, 0))],
        out_specs=pl.BlockSpec((CHUNK, 1), lambda kc: (kc, 0)),
        out_shape=jax.ShapeDtypeStruct((KPAD, 1), jnp.float32),
    )(kpad)


def _dist_body(q_ref, k_ref, qsq_ref, ksq_ref, out_ref, m16_ref, cm_ref):
    qblk = q_ref[...]
    kblk = k_ref[...]
    dots = jax.lax.dot_general(
        qblk.astype(jnp.bfloat16), kblk.astype(jnp.bfloat16),
        (((1,), (1,)), ((), ())),
        preferred_element_type=jnp.float32)
    dists = (qsq_ref[...] + ksq_ref[...]) - 2.0 * dots
    out_ref[...] = dists
    m16 = dists[:, 0:128]
    for s in range(1, 16):
        m16 = jnp.minimum(m16, dists[:, s * 128:(s + 1) * 128])
    m16_ref[0] = m16
    mins = [jnp.min(m16[:, r * 32:(r + 1) * 32], axis=1, keepdims=True)
            for r in range(4)]
    cm_ref[0] = jnp.concatenate(mins, axis=1)


def _dists(q, kpad, qsq, ksq_row):
    return pl.pallas_call(
        _dist_body,
        grid=(KPAD // CHUNK, Q // QB),
        in_specs=[
            pl.BlockSpec((QB, D), lambda kc, qb: (qb, 0)),
            pl.BlockSpec((CHUNK, D), lambda kc, qb: (kc, 0)),
            pl.BlockSpec((QB, 1), lambda kc, qb: (qb, 0)),
            pl.BlockSpec((1, CHUNK), lambda kc, qb: (0, kc)),
        ],
        out_specs=[
            pl.BlockSpec((QB, CHUNK), lambda kc, qb: (qb, kc)),
            pl.BlockSpec((1, QB, 128), lambda kc, qb: (kc, qb, 0)),
            pl.BlockSpec((1, QB, CHUNK // SUB), lambda kc, qb: (kc, qb, 0)),
        ],
        out_shape=[
            jax.ShapeDtypeStruct((Q, KPAD), jnp.float32),
            jax.ShapeDtypeStruct((KPAD // CHUNK, Q, 128), jnp.float32),
            jax.ShapeDtypeStruct((KPAD // CHUNK, Q, CHUNK // SUB),
                                 jnp.float32),
        ],
    )(q, kpad, qsq, ksq_row)


def _lo_body(cm_ref, out_ref):
    a = cm_ref[...]
    m = jnp.min(a, axis=1, keepdims=True)
    for _ in range(TOPK - 1):
        a = jnp.where(a == m, INF, a)
        m = jnp.min(a, axis=1, keepdims=True)
    out_ref[...] = m


def _lo(cm):
    return pl.pallas_call(
        _lo_body,
        grid=(Q // QB,),
        in_specs=[pl.BlockSpec((QB, NSUBPAD), lambda qb: (qb, 0))],
        out_specs=pl.BlockSpec((QB, 1), lambda qb: (qb, 0)),
        out_shape=jax.ShapeDtypeStruct((Q, 1), jnp.float32),
    )(cm)


def _sc_compact(sflat, m16, upb):
    info = plsc.get_sparse_core_info()
    mesh = plsc.VectorSubcoreMesh(core_axis_name="c", subcore_axis_name="s")

    @functools.partial(
        pl.kernel, mesh=mesh,
        compiler_params=pltpu.CompilerParams(needs_layout_passes=False),
        out_type=[
            jax.ShapeDtypeStruct((Q, B), jnp.float32),
            jax.ShapeDtypeStruct((Q, B), jnp.int32),
        ],
        scratch_types=[
            pltpu.VMEM((NG16,), jnp.float32),
            pltpu.VMEM((BPAD,), jnp.int32),
            pltpu.VMEM((BPAD * 16,), jnp.float32),
            pltpu.VMEM((BPAD,), jnp.float32),
            pltpu.VMEM((BPAD,), jnp.int32),
            pltpu.VMEM((16,), jnp.float32),
            pltpu.SemaphoreType.DMA,
            pltpu.SemaphoreType.DMA,
        ],
    )
    def body(s_hbm, m16_hbm, up_hbm, cv_hbm, ci_hbm,
             m16_v, gl_v, gb_v, cv_v, ci_v, up_v, sem, gsem):
        wid = lax.axis_index("s") * info.num_cores + lax.axis_index("c")
        q0 = wid * QPW
        iota = lax.iota(jnp.int32, 16)

        def per_query(t, _):
            qi = q0 + t
            pltpu.sync_copy(up_hbm.at[qi], up_v)
            upb = up_v[...]
            pltpu.sync_copy(m16_hbm.at[qi], m16_v)
            neg = jnp.full((16,), INF, jnp.float32)
            zer = jnp.zeros((16,), jnp.int32)

            def init(b16, _):
                cv_v[pl.ds(b16 * 16, 16)] = neg
                ci_v[pl.ds(b16 * 16, 16)] = zer
                return 0

            lax.fori_loop(0, BPAD // 16, init, 0)

            def scan_m16(i, ng):
                v = m16_v[pl.ds(i * 16, 16)]
                m = v <= upb
                c = plsc.cumsum(jnp.where(m, 1, 0).astype(jnp.int32))
                pos = ng + c - 1
                plsc.store_scatter(gl_v, [pos], iota + i * 16, mask=m)
                cnt = plsc.all_reduce_population_count(m)[0]
                return jnp.minimum(ng + cnt, B)

            ng = lax.fori_loop(0, NG16 // 16, scan_m16, jnp.int32(0))

            qflat = qi * KPAD

            def fire(j, _):
                gid = plsc.load_gather(
                    gl_v, [jnp.broadcast_to(j, (16,)).astype(jnp.int32)])[0]
                base = qflat + (gid // 128) * CHUNK + (gid % 128)
                idxv = base + iota * 128
                pltpu.async_copy(
                    s_hbm.at[idxv], gb_v.at[pl.ds(j * 16, 16)], gsem)
                return 0

            lax.fori_loop(0, ng, fire, 0)

            def drain(j, _):
                pltpu.make_async_copy(
                    s_hbm.at[iota], gb_v.at[pl.ds(0, 16)], gsem).wait()
                return 0

            lax.fori_loop(0, ng, drain, 0)

            def scan_groups(j, off):
                gid = plsc.load_gather(
                    gl_v, [jnp.broadcast_to(j, (16,)).astype(jnp.int32)])[0]
                v = gb_v[pl.ds(j * 16, 16)]
                m = v <= upb
                idxv = (gid // 128) * CHUNK + (gid % 128) + iota * 128
                c = plsc.cumsum(jnp.where(m, 1, 0).astype(jnp.int32))
                pos = off + c - 1
                plsc.store_scatter(cv_v, [pos], v, mask=m)
                plsc.store_scatter(ci_v, [pos], idxv, mask=m)
                cnt = plsc.all_reduce_population_count(m)[0]
                return jnp.minimum(off + cnt, B)

            lax.fori_loop(0, ng, scan_groups, jnp.int32(0))
            pltpu.sync_copy(cv_v.at[pl.ds(0, B)], cv_hbm.at[qi])
            pltpu.sync_copy(ci_v.at[pl.ds(0, B)], ci_hbm.at[qi])
            return 0

        lax.fori_loop(0, QPW, per_query, 0)

    return body(sflat, m16, upb)


def _final_body(cv_ref, ci_ref, out_ref):
    a = cv_ref[...]
    idx = ci_ref[...]
    outs = []
    for _ in range(TOPK):
        m = jnp.min(a, axis=1, keepdims=True)
        sel = a == m
        cand = jnp.max(jnp.where(sel, idx, -1), axis=1, keepdims=True)
        outs.append(cand)
        a = jnp.where(sel & (idx == cand), INF, a)
    out_ref[...] = jnp.concatenate(outs, axis=1)


def _final(cv, ci):
    return pl.pallas_call(
        _final_body,
        grid=(Q // QB,),
        in_specs=[
            pl.BlockSpec((QB, B), lambda qb: (qb, 0)),
            pl.BlockSpec((QB, B), lambda qb: (qb, 0)),
        ],
        out_specs=pl.BlockSpec((QB, TOPK), lambda qb: (qb, 0)),
        out_shape=jax.ShapeDtypeStruct((Q, TOPK), jnp.int32),
    )(cv, ci)


def kernel(input_embeddings, index_embeddings, top_k):
    kpad = jnp.concatenate(
        [index_embeddings,
         jnp.full((KPAD - KDB, D), 1e4, jnp.float32)], axis=0)
    qsq = jnp.sum(input_embeddings * input_embeddings, axis=1, keepdims=True)
    ksq_row = _ksq(kpad).reshape(1, KPAD)
    dists, m163, cm3 = _dists(input_embeddings, kpad, qsq, ksq_row)
    cm = jnp.transpose(cm3, (1, 0, 2)).reshape(Q, NSUB)
    cm = jnp.concatenate(
        [cm, jnp.full((Q, NSUBPAD - NSUB), jnp.inf, jnp.float32)], axis=1)
    up = _lo(cm)
    upb = jnp.broadcast_to(up, (Q, 16))
    m16 = jnp.transpose(m163, (1, 0, 2)).reshape(Q, NG16)
    cv, ci = _sc_compact(dists.reshape(Q * KPAD), m16, upb)
    return _final(cv, ci)


# speedup vs baseline: 12.4112x; 1.0843x over previous
import functools

import jax
import jax.numpy as jnp
from jax import lax
from jax.experimental import pallas as pl
from jax.experimental.pallas import tpu as pltpu
from jax.experimental.pallas import tpu_sc as plsc

Q = 1024
D = 64
KDB = 100000
KPAD = 100352
CHUNK = 2048
SUB = 512
NSUB = KPAD // SUB
NSUBPAD = 256
QB = 256
TOPK = 50
B = 256
BPAD = B + 16
NG16 = KPAD // 16
QPW = 32
INF = float('inf')


def _ksq_body(k_ref, out_ref):
    kblk = k_ref[...]
    out_ref[...] = jnp.sum(kblk * kblk, axis=1, keepdims=True)


def _ksq(kpad):
    return pl.pallas_call(
        _ksq_body,
        grid=(KPAD // CHUNK,),
        in_specs=[pl.BlockSpec((CHUNK, D), lambda kc: (kc, 0))],
        out_specs=pl.BlockSpec((CHUNK, 1), lambda kc: (kc, 0)),
        out_shape=jax.ShapeDtypeStruct((KPAD, 1), jnp.float32),
    )(kpad)


def _dist_body(q_ref, k_ref, qsq_ref, ksq_ref, out_ref, m16_ref):
    qblk = q_ref[...]
    kblk = k_ref[...]
    dots = jax.lax.dot_general(
        qblk.astype(jnp.bfloat16), kblk.astype(jnp.bfloat16),
        (((1,), (1,)), ((), ())),
        preferred_element_type=jnp.float32)
    dists = (qsq_ref[...] + ksq_ref[...]) - 2.0 * dots
    out_ref[...] = dists
    m16 = dists[:, 0:128]
    for s in range(1, 16):
        m16 = jnp.minimum(m16, dists[:, s * 128:(s + 1) * 128])
    m16_ref[...] = m16


def _dists(q, kpad, qsq, ksq_row):
    return pl.pallas_call(
        _dist_body,
        grid=(KPAD // CHUNK, Q // QB),
        in_specs=[
            pl.BlockSpec((QB, D), lambda kc, qb: (qb, 0)),
            pl.BlockSpec((CHUNK, D), lambda kc, qb: (kc, 0)),
            pl.BlockSpec((QB, 1), lambda kc, qb: (qb, 0)),
            pl.BlockSpec((1, CHUNK), lambda kc, qb: (0, kc)),
        ],
        out_specs=[
            pl.BlockSpec((QB, CHUNK), lambda kc, qb: (qb, kc)),
            pl.BlockSpec((QB, 128), lambda kc, qb: (qb, kc)),
        ],
        out_shape=[
            jax.ShapeDtypeStruct((Q, KPAD), jnp.float32),
            jax.ShapeDtypeStruct((Q, NG16), jnp.float32),
        ],
    )(q, kpad, qsq, ksq_row)


def _lo_body(cm_ref, out_ref):
    a = cm_ref[...]
    m = jnp.min(a, axis=1, keepdims=True)
    for _ in range(TOPK - 1):
        a = jnp.where(a == m, INF, a)
        m = jnp.min(a, axis=1, keepdims=True)
    out_ref[...] = m


def _lo(cm):
    return pl.pallas_call(
        _lo_body,
        grid=(Q // QB,),
        in_specs=[pl.BlockSpec((QB, NG16), lambda qb: (qb, 0))],
        out_specs=pl.BlockSpec((QB, 1), lambda qb: (qb, 0)),
        out_shape=jax.ShapeDtypeStruct((Q, 1), jnp.float32),
    )(cm)


def _sc_compact(sflat, m16, upb):
    info = plsc.get_sparse_core_info()
    mesh = plsc.VectorSubcoreMesh(core_axis_name="c", subcore_axis_name="s")

    @functools.partial(
        pl.kernel, mesh=mesh,
        compiler_params=pltpu.CompilerParams(needs_layout_passes=False),
        out_type=[
            jax.ShapeDtypeStruct((Q, B), jnp.float32),
            jax.ShapeDtypeStruct((Q, B), jnp.int32),
        ],
        scratch_types=[
            pltpu.VMEM((NG16,), jnp.float32),
            pltpu.VMEM((BPAD,), jnp.int32),
            pltpu.VMEM((BPAD * 16,), jnp.float32),
            pltpu.VMEM((BPAD,), jnp.float32),
            pltpu.VMEM((BPAD,), jnp.int32),
            pltpu.VMEM((16,), jnp.float32),
            pltpu.SemaphoreType.DMA,
            pltpu.SemaphoreType.DMA,
        ],
    )
    def body(s_hbm, m16_hbm, up_hbm, cv_hbm, ci_hbm,
             m16_v, gl_v, gb_v, cv_v, ci_v, up_v, sem, gsem):
        wid = lax.axis_index("s") * info.num_cores + lax.axis_index("c")
        q0 = wid * QPW
        iota = lax.iota(jnp.int32, 16)

        def per_query(t, _):
            qi = q0 + t
            pltpu.sync_copy(up_hbm.at[qi], up_v)
            upb = up_v[...]
            pltpu.sync_copy(m16_hbm.at[qi], m16_v)
            neg = jnp.full((16,), INF, jnp.float32)
            zer = jnp.zeros((16,), jnp.int32)

            def init(b16, _):
                cv_v[pl.ds(b16 * 16, 16)] = neg
                ci_v[pl.ds(b16 * 16, 16)] = zer
                return 0

            lax.fori_loop(0, BPAD // 16, init, 0)

            def scan_m16(i, ng):
                v = m16_v[pl.ds(i * 16, 16)]
                m = v <= upb
                c = plsc.cumsum(jnp.where(m, 1, 0).astype(jnp.int32))
                pos = ng + c - 1
                plsc.store_scatter(gl_v, [pos], iota + i * 16, mask=m)
                cnt = plsc.all_reduce_population_count(m)[0]
                return jnp.minimum(ng + cnt, B)

            ng = lax.fori_loop(0, NG16 // 16, scan_m16, jnp.int32(0))

            qflat = qi * KPAD

            def fire(j, _):
                gid = plsc.load_gather(
                    gl_v, [jnp.broadcast_to(j, (16,)).astype(jnp.int32)])[0]
                base = qflat + (gid // 128) * CHUNK + (gid % 128)
                idxv = base + iota * 128
                pltpu.async_copy(
                    s_hbm.at[idxv], gb_v.at[pl.ds(j * 16, 16)], gsem)
                return 0

            lax.fori_loop(0, ng, fire, 0)

            def drain(j, _):
                pltpu.make_async_copy(
                    s_hbm.at[iota], gb_v.at[pl.ds(0, 16)], gsem).wait()
                return 0

            lax.fori_loop(0, ng, drain, 0)

            def scan_groups(j, off):
                gid = plsc.load_gather(
                    gl_v, [jnp.broadcast_to(j, (16,)).astype(jnp.int32)])[0]
                v = gb_v[pl.ds(j * 16, 16)]
                m = v <= upb
                idxv = (gid // 128) * CHUNK + (gid % 128) + iota * 128
                c = plsc.cumsum(jnp.where(m, 1, 0).astype(jnp.int32))
                pos = off + c - 1
                plsc.store_scatter(cv_v, [pos], v, mask=m)
                plsc.store_scatter(ci_v, [pos], idxv, mask=m)
                cnt = plsc.all_reduce_population_count(m)[0]
                return jnp.minimum(off + cnt, B)

            lax.fori_loop(0, ng, scan_groups, jnp.int32(0))
            pltpu.sync_copy(cv_v.at[pl.ds(0, B)], cv_hbm.at[qi])
            pltpu.sync_copy(ci_v.at[pl.ds(0, B)], ci_hbm.at[qi])
            return 0

        lax.fori_loop(0, QPW, per_query, 0)

    return body(sflat, m16, upb)


def _final_body(cv_ref, ci_ref, out_ref):
    a = cv_ref[...]
    idx = ci_ref[...]
    outs = []
    for _ in range(TOPK):
        m = jnp.min(a, axis=1, keepdims=True)
        sel = a == m
        cand = jnp.max(jnp.where(sel, idx, -1), axis=1, keepdims=True)
        outs.append(cand)
        a = jnp.where(sel & (idx == cand), INF, a)
    out_ref[...] = jnp.concatenate(outs, axis=1)


def _final(cv, ci):
    return pl.pallas_call(
        _final_body,
        grid=(Q // QB,),
        in_specs=[
            pl.BlockSpec((QB, B), lambda qb: (qb, 0)),
            pl.BlockSpec((QB, B), lambda qb: (qb, 0)),
        ],
        out_specs=pl.BlockSpec((QB, TOPK), lambda qb: (qb, 0)),
        out_shape=jax.ShapeDtypeStruct((Q, TOPK), jnp.int32),
    )(cv, ci)


def kernel(input_embeddings, index_embeddings, top_k):
    kpad = jnp.concatenate(
        [index_embeddings,
         jnp.full((KPAD - KDB, D), 1e4, jnp.float32)], axis=0)
    qsq = jnp.sum(input_embeddings * input_embeddings, axis=1, keepdims=True)
    ksq_row = _ksq(kpad).reshape(1, KPAD)
    dists, m16 = _dists(input_embeddings, kpad, qsq, ksq_row)
    up = _lo(m16)
    upb = jnp.broadcast_to(up, (Q, 16))
    cv, ci = _sc_compact(dists.reshape(Q * KPAD), m16, upb)
    return _final(cv, ci)
